# baseline (speedup 1.0000x reference)
import jax
import jax.numpy as jnp
from jax import lax
from jax.experimental import pallas as pl
from jax.experimental.pallas import tpu as pltpu, tpu_sc as plsc

WIDTH = 1000
FEATURE_DIM = 100000
N = 1024
ZSH = 400000
HALF = N * FEATURE_DIM // 2
NCHUNK = HALF // ZSH


def _scs_body(zeros_hbm, state_hbm, out_hbm, zsh, sem, zsem):
    c = lax.axis_index("c")
    base = c * HALF

    pltpu.sync_copy(zeros_hbm, zsh)

    def floop(i, carry):
        pltpu.make_async_copy(
            zsh, out_hbm.at[pl.ds(base + i * ZSH, ZSH)], zsem).start()
        return carry

    lax.fori_loop(0, NCHUNK, floop, 0)

    def wloop(i, carry):
        pltpu.make_async_copy(
            zsh, out_hbm.at[pl.ds(base + i * ZSH, ZSH)], zsem).wait()
        return carry

    lax.fori_loop(0, NCHUNK, wloop, 0)


def kernel(state):
    n = state.shape[0]
    zeros_flat = jnp.zeros((ZSH,), jnp.float32)
    out = pl.kernel(
        _scs_body,
        out_type=jax.ShapeDtypeStruct((n * FEATURE_DIM,), jnp.float32),
        mesh=plsc.ScalarSubcoreMesh(axis_name="c", num_cores=2),
        scratch_types=[
            pltpu.VMEM_SHARED((ZSH,), jnp.float32),
            pltpu.SemaphoreType.DMA,
            pltpu.SemaphoreType.DMA,
        ],
    )(zeros_flat, state.reshape(-1))
    return out.reshape(n, FEATURE_DIM)


# speedup vs baseline: 2.2866x; 2.2866x over previous
import jax
import jax.numpy as jnp
from jax.experimental import pallas as pl

WIDTH = 1000
FEATURE_DIM = 100000
ROW_BLOCK = 16


def _onehot_block(state_ref, out_ref):
    idx = state_ref[:, 0] + WIDTH * state_ref[:, 1]
    cols = jax.lax.broadcasted_iota(jnp.int32, out_ref.shape, 1)
    out_ref[...] = (cols == idx[:, None]).astype(jnp.float32)


def kernel(state):
    n = state.shape[0]
    grid = n // ROW_BLOCK
    return pl.pallas_call(
        _onehot_block,
        grid=(grid,),
        in_specs=[pl.BlockSpec((ROW_BLOCK, 2), lambda i: (i, 0))],
        out_specs=pl.BlockSpec((ROW_BLOCK, FEATURE_DIM), lambda i: (i, 0)),
        out_shape=jax.ShapeDtypeStruct((n, FEATURE_DIM), jnp.float32),
    )(state)
